# Initial kernel scaffold; baseline (speedup 1.0000x reference)
import jax
import jax.numpy as jnp
from jax.experimental import pallas as pl


def kernel(features, edge_index, W0, b0, W1, b1, W2, b2):
    raise NotImplementedError("write your pallas kernel here")


# speedup vs baseline: 7.0427x; 7.0427x over previous
import functools

import jax
import jax.numpy as jnp
from jax import lax
from jax.experimental import pallas as pl
from jax.experimental.pallas import tpu as pltpu
from jax.experimental.pallas import tpu_sc as plsc

N = 10000
E = 320000
D_IN = 128
D_H = 128
D_OUT = 64

NC = 2
NS = 16
NW = NC * NS
CHUNK = 128
CPW = 79
E_PAD = NW * CPW * CHUNK
N_PAD = 10112
RPT_PAD = N_PAD // NS

_mesh = plsc.VectorSubcoreMesh(core_axis_name="c", subcore_axis_name="s")


@functools.partial(
    pl.kernel,
    mesh=_mesh,
    out_type=jax.ShapeDtypeStruct((NC, N_PAD, D_H), jnp.float32),
    scratch_types=[
        pltpu.VMEM_SHARED((N_PAD, D_H), jnp.float32),
        pltpu.VMEM((CPW, CHUNK), jnp.int32),
        pltpu.VMEM((CHUNK, D_H), jnp.float32),
    ],
)
def _hist_kernel(idx_hbm, ones_hbm, zeros_hbm, deg_hbm, sh, idx_v, ones_v):
    c = lax.axis_index("c")
    s = lax.axis_index("s")
    wid = c * NS + s
    pltpu.sync_copy(idx_hbm.at[wid], idx_v)
    pltpu.sync_copy(ones_hbm, ones_v)
    z0 = s * RPT_PAD
    pltpu.sync_copy(zeros_hbm, sh.at[pl.ds(z0, RPT_PAD)])
    plsc.subcore_barrier()

    def body(j, carry):
        pltpu.sync_copy(ones_v, sh.at[idx_v.at[j]], add=True)
        return carry

    lax.fori_loop(0, CPW, body, 0)
    plsc.subcore_barrier()
    pltpu.sync_copy(sh.at[pl.ds(z0, RPT_PAD)],
                    deg_hbm.at[c, pl.ds(z0, RPT_PAD)])


def _make_agg(D):
    @functools.partial(
        pl.kernel,
        mesh=_mesh,
        out_type=jax.ShapeDtypeStruct((NC, N_PAD, D), jnp.float32),
        scratch_types=[
            pltpu.VMEM_SHARED((N_PAD, D), jnp.float32),
            pltpu.VMEM((CPW, CHUNK), jnp.int32),
            pltpu.VMEM((CPW, CHUNK), jnp.int32),
            pltpu.VMEM((CHUNK, D), jnp.float32),
            pltpu.SemaphoreType.DMA,
        ],
    )
    def _agg(hw_hbm, src_hbm, dst_hbm, zeros_hbm,
             out_hbm, sh, isrc, idst, buf, sem):
        c = lax.axis_index("c")
        s = lax.axis_index("s")
        wid = c * NS + s
        pltpu.sync_copy(src_hbm.at[wid], isrc)
        pltpu.sync_copy(dst_hbm.at[wid], idst)
        z0 = s * RPT_PAD
        pltpu.sync_copy(zeros_hbm, sh.at[pl.ds(z0, RPT_PAD)])
        plsc.subcore_barrier()

        def body(j, carry):
            pltpu.async_copy(hw_hbm.at[isrc.at[j]], buf, sem).wait()
            pltpu.sync_copy(buf, sh.at[idst.at[j]], add=True)
            return carry

        lax.fori_loop(0, CPW, body, 0)
        plsc.subcore_barrier()
        pltpu.sync_copy(sh.at[pl.ds(z0, RPT_PAD)],
                        out_hbm.at[c, pl.ds(z0, RPT_PAD)])

    return _agg


_agg128 = _make_agg(D_H)


def _tc_first_body(f_ref, w_ref, dego_ref, degi_ref, hw_ref, ns_ref, nd_ref):
    deg_o = (dego_ref[0] + dego_ref[1])[:N, 0:1]
    deg_i = (degi_ref[0] + degi_ref[1])[:N, 0:1]
    ns = jnp.where(deg_o > 0.0, lax.rsqrt(jnp.maximum(deg_o, 1.0)), 0.0)
    nd = jnp.where(deg_i > 0.0, lax.rsqrt(jnp.maximum(deg_i, 1.0)), 0.0)
    ns_ref[...] = ns
    nd_ref[...] = nd
    hw_ref[...] = jnp.dot(f_ref[...], w_ref[...],
                          preferred_element_type=jnp.float32) * ns


_tc_first = pl.pallas_call(
    _tc_first_body,
    out_shape=(
        jax.ShapeDtypeStruct((N, D_H), jnp.float32),
        jax.ShapeDtypeStruct((N, 1), jnp.float32),
        jax.ShapeDtypeStruct((N, 1), jnp.float32),
    ),
)


def _tc_mid_body(p_ref, ns_ref, nd_ref, b_ref, w_ref, o_ref):
    h = (p_ref[0] + p_ref[1])[:N] * nd_ref[...] + b_ref[...]
    h = jnp.maximum(h, 0.0)
    o_ref[...] = jnp.dot(h, w_ref[...],
                         preferred_element_type=jnp.float32) * ns_ref[...]


_tc_mid = pl.pallas_call(
    _tc_mid_body,
    out_shape=jax.ShapeDtypeStruct((N, D_H), jnp.float32),
)


def _tc_last_body(p_ref, nd_ref, b_ref, o_ref):
    o_ref[...] = (p_ref[0] + p_ref[1])[:N, :D_OUT] * nd_ref[...] + b_ref[...]


_tc_last = pl.pallas_call(
    _tc_last_body,
    out_shape=jax.ShapeDtypeStruct((N, D_OUT), jnp.float32),
)


@jax.jit
def kernel(features, edge_index, W0, b0, W1, b1, W2, b2):
    src = edge_index[0]
    dst = edge_index[1]
    pad = E_PAD - E
    trash = N + (jnp.arange(pad, dtype=jnp.int32) % (N_PAD - N))
    spread = jnp.arange(pad, dtype=jnp.int32) % N
    src_deg = jnp.concatenate([src, trash]).reshape(NW, CPW, CHUNK)
    dst_any = jnp.concatenate([dst, trash]).reshape(NW, CPW, CHUNK)
    src_gat = jnp.concatenate([src, spread]).reshape(NW, CPW, CHUNK)

    ones128 = jnp.ones((CHUNK, D_H), jnp.float32)
    zer128 = jnp.zeros((RPT_PAD, D_H), jnp.float32)
    W2p = jnp.pad(W2, ((0, 0), (0, D_H - D_OUT)))

    dego = _hist_kernel(src_deg, ones128, zer128)
    degi = _hist_kernel(dst_any, ones128, zer128)
    hw0, ns, nd = _tc_first(features, W0, dego, degi)
    p0 = _agg128(hw0, src_gat, dst_any, zer128)
    hw1 = _tc_mid(p0, ns, nd, b0.reshape(1, D_H), W1)
    p1 = _agg128(hw1, src_gat, dst_any, zer128)
    hw2 = _tc_mid(p1, ns, nd, b1.reshape(1, D_H), W2p)
    p2 = _agg128(hw2, src_gat, dst_any, zer128)
    out = _tc_last(p2, nd, b2.reshape(1, D_OUT))
    return out
